# baseline (speedup 1.0000x reference)
import math

import jax
import jax.numpy as jnp
from jax import lax
from jax.experimental import pallas as pl
from jax.experimental.pallas import tpu as pltpu
from jax.experimental.pallas import tpu_sc as plsc

D_MODEL = 1024
MAX_LEN = 8192
BATCH = 4
SEQ = 2048
SCALE = math.sqrt(D_MODEL)

NC = 2
NS = 16
NW = NC * NS
ROWS = BATCH * SEQ
ROWS_PER_W = ROWS // NW
CH = 16
NCHUNK = ROWS_PER_W // CH
U = 8
LANES = 16


def _body(x_hbm, idx_hbm, pe_hbm, out_hbm, pos_hbm, idx_v, pe_buf, x_buf, sem):
    wid = lax.axis_index("c") * NS + lax.axis_index("s")
    row0 = wid * ROWS_PER_W

    pltpu.sync_copy(idx_hbm.at[pl.ds(row0, ROWS_PER_W)], idx_v)

    def chunk(i, carry):
        dst = row0 + i * CH
        pltpu.async_copy(
            pe_hbm.at[idx_v.at[pl.ds(i * CH, CH)]], pe_buf, sem
        ).wait()
        pltpu.sync_copy(x_hbm.at[pl.ds(dst, CH), :], x_buf)
        pltpu.sync_copy(pe_buf, pos_hbm.at[pl.ds(dst, CH), :])

        def row(r, c1):
            def grp(g, c2):
                base = g * (LANES * U)
                for u in range(U):
                    s = pl.ds(base + u * LANES, LANES)
                    x_buf[r, s] = x_buf[r, s] * SCALE + pe_buf[r, s]
                return c2

            return lax.fori_loop(0, D_MODEL // (LANES * U), grp, c1)

        lax.fori_loop(0, CH, row, 0)
        pltpu.sync_copy(x_buf, out_hbm.at[pl.ds(dst, CH), :])
        return carry

    lax.fori_loop(0, NCHUNK, chunk, 0)


@jax.jit
def _sc_call(x2d, idx, pe):
    mesh = plsc.VectorSubcoreMesh(core_axis_name="c", subcore_axis_name="s")
    return pl.kernel(
        _body,
        out_type=(
            jax.ShapeDtypeStruct((ROWS, D_MODEL), jnp.float32),
            jax.ShapeDtypeStruct((ROWS, D_MODEL), jnp.float32),
        ),
        mesh=mesh,
        scratch_types=[
            pltpu.VMEM((ROWS_PER_W,), jnp.int32),
            pltpu.VMEM((CH, D_MODEL), jnp.float32),
            pltpu.VMEM((CH, D_MODEL), jnp.float32),
            pltpu.SemaphoreType.DMA,
        ],
    )(x2d, idx, pe)


def kernel(x, offset, pe):
    assert x.shape == (BATCH, SEQ, D_MODEL)
    x2d = x.reshape(ROWS, D_MODEL)
    index = offset[:, None].astype(jnp.int32) + jnp.arange(SEQ, dtype=jnp.int32)
    index = jnp.maximum(index, 0).reshape(ROWS)
    out_f, pos_f = _sc_call(x2d, index, pe)
    return out_f.reshape(x.shape), pos_f.reshape(x.shape)


# speedup vs baseline: 2.6311x; 2.6311x over previous
import math

import jax
import jax.numpy as jnp
from jax import lax
from jax.experimental import pallas as pl
from jax.experimental.pallas import tpu as pltpu
from jax.experimental.pallas import tpu_sc as plsc

D_MODEL = 1024
MAX_LEN = 8192
BATCH = 4
SEQ = 2048
SCALE = math.sqrt(D_MODEL)

NC = 2
NS = 16
NW = NC * NS
ROWS = BATCH * SEQ
ROWS_PER_W = ROWS // NW
CH = 8
NCHUNK = ROWS_PER_W // CH
NSLOT = 4
G = NCHUNK // NSLOT
U = 8
LANES = 16


def _body(x_hbm, idx_hbm, pe_hbm, out_hbm, pos_hbm,
          idx_v, pe_b, x_b, o_b, sem_in, sem_out):
    wid = lax.axis_index("c") * NS + lax.axis_index("s")
    row0 = wid * ROWS_PER_W

    pltpu.sync_copy(idx_hbm.at[pl.ds(row0, ROWS_PER_W)], idx_v)

    def start_in(i, k):
        pltpu.async_copy(pe_hbm.at[idx_v.at[pl.ds(i * CH, CH)]],
                         pe_b[k], sem_in[k])
        pltpu.async_copy(x_hbm.at[pl.ds(row0 + i * CH, CH), :],
                         x_b[k], sem_in[k])

    def wait_in(i, k):
        pltpu.make_async_copy(pe_hbm.at[idx_v.at[pl.ds(i * CH, CH)]],
                              pe_b[k], sem_in[k]).wait()
        pltpu.make_async_copy(x_hbm.at[pl.ds(row0 + i * CH, CH), :],
                              x_b[k], sem_in[k]).wait()

    for k in range(NSLOT):
        start_in(k, k)

    def outer(g, carry):
        for k in range(NSLOT):
            i = g * NSLOT + k
            wait_in(i, k)
            pltpu.async_copy(pe_b[k], pos_hbm.at[pl.ds(row0 + i * CH, CH), :],
                             sem_out[k])

            @pl.when(g > 0)
            def _drain_out():
                pltpu.make_async_copy(
                    o_b[k], out_hbm.at[pl.ds(row0 + i * CH, CH), :],
                    sem_out[k]).wait()

            for r in range(CH):
                def grp(c, c2, _r=r):
                    base = c * (LANES * U)
                    for u in range(U):
                        s = pl.ds(base + u * LANES, LANES)
                        o_b[k][_r, s] = x_b[k][_r, s] * SCALE + pe_b[k][_r, s]
                    return c2
                lax.fori_loop(0, D_MODEL // (LANES * U), grp, 0)

            pltpu.async_copy(o_b[k], out_hbm.at[pl.ds(row0 + i * CH, CH), :],
                             sem_out[k])

            pltpu.make_async_copy(
                pe_b[k], pos_hbm.at[pl.ds(row0 + i * CH, CH), :],
                sem_out[k]).wait()

            @pl.when(g < G - 1)
            def _prefetch():
                start_in(i + NSLOT, k)
        return carry

    lax.fori_loop(0, G, outer, 0)

    for k in range(NSLOT):
        i = (G - 1) * NSLOT + k
        pltpu.make_async_copy(o_b[k], out_hbm.at[pl.ds(row0 + i * CH, CH), :],
                              sem_out[k]).wait()


def _body_wrap(x_hbm, idx_hbm, pe_hbm, out_hbm, pos_hbm,
               idx_v, pe0, pe1, pe2, pe3, x0, x1, x2, x3,
               o0, o1, o2, o3, si0, si1, si2, si3, so0, so1, so2, so3):
    _body(x_hbm, idx_hbm, pe_hbm, out_hbm, pos_hbm, idx_v,
          (pe0, pe1, pe2, pe3), (x0, x1, x2, x3), (o0, o1, o2, o3),
          (si0, si1, si2, si3), (so0, so1, so2, so3))


@jax.jit
def _sc_call(x2d, idx, pe):
    mesh = plsc.VectorSubcoreMesh(core_axis_name="c", subcore_axis_name="s")
    buf = pltpu.VMEM((CH, D_MODEL), jnp.float32)
    return pl.kernel(
        _body_wrap,
        out_type=(
            jax.ShapeDtypeStruct((ROWS, D_MODEL), jnp.float32),
            jax.ShapeDtypeStruct((ROWS, D_MODEL), jnp.float32),
        ),
        mesh=mesh,
        scratch_types=(
            [pltpu.VMEM((ROWS_PER_W,), jnp.int32)]
            + [buf] * (3 * NSLOT)
            + [pltpu.SemaphoreType.DMA] * (2 * NSLOT)
        ),
    )(x2d, idx, pe)


def kernel(x, offset, pe):
    assert x.shape == (BATCH, SEQ, D_MODEL)
    x2d = x.reshape(ROWS, D_MODEL)
    index = offset[:, None].astype(jnp.int32) + jnp.arange(SEQ, dtype=jnp.int32)
    index = jnp.maximum(index, 0).reshape(ROWS)
    out_f, pos_f = _sc_call(x2d, index, pe)
    return out_f.reshape(x.shape), pos_f.reshape(x.shape)


# speedup vs baseline: 2.7064x; 1.0286x over previous
import math

import jax
import jax.numpy as jnp
from jax import lax
from jax.experimental import pallas as pl
from jax.experimental.pallas import tpu as pltpu
from jax.experimental.pallas import tpu_sc as plsc

D_MODEL = 1024
MAX_LEN = 8192
BATCH = 4
SEQ = 2048
SCALE = math.sqrt(D_MODEL)

NC = 2
NS = 16
NW = NC * NS
ROWS = BATCH * SEQ
ROWS_PER_W = ROWS // NW
CH = 8
NCHUNK = ROWS_PER_W // CH
NSLOT = 4
G = NCHUNK // NSLOT
U = 8
LANES = 16


def _body(x_hbm, idx_hbm, pe_hbm, out_hbm, pos_hbm,
          idx_v, pe_b, x_b, o_b, sem_in, sem_out):
    wid = lax.axis_index("c") * NS + lax.axis_index("s")
    row0 = wid * ROWS_PER_W

    pltpu.sync_copy(idx_hbm.at[pl.ds(row0, ROWS_PER_W)], idx_v)

    def start_in(i, k):
        pltpu.async_copy(pe_hbm.at[idx_v.at[pl.ds(i * CH, CH)]],
                         pe_b[k], sem_in[k])
        pltpu.async_copy(x_hbm.at[pl.ds(row0 + i * CH, CH), :],
                         x_b[k], sem_in[k])

    def wait_in(i, k):
        pltpu.make_async_copy(pe_hbm.at[idx_v.at[pl.ds(i * CH, CH)]],
                              pe_b[k], sem_in[k]).wait()
        pltpu.make_async_copy(x_hbm.at[pl.ds(row0 + i * CH, CH), :],
                              x_b[k], sem_in[k]).wait()

    for k in range(NSLOT):
        start_in(k, k)

    def outer(g, carry):
        for k in range(NSLOT):
            i = g * NSLOT + k
            wait_in(i, k)
            pltpu.async_copy(pe_b[k], pos_hbm.at[pl.ds(row0 + i * CH, CH), :],
                             sem_out[k])

            @pl.when(g > 0)
            def _drain_out():
                pltpu.make_async_copy(
                    o_b[k], out_hbm.at[pl.ds(row0 + i * CH, CH), :],
                    sem_out[k]).wait()

            for r in range(CH):
                @plsc.parallel_loop(0, D_MODEL // LANES, step=1, unroll=U)
                def _grp(c, _r=r, _k=k):
                    s = pl.ds(c * LANES, LANES)
                    o_b[_k][_r, s] = x_b[_k][_r, s] * SCALE + pe_b[_k][_r, s]

            pltpu.async_copy(o_b[k], out_hbm.at[pl.ds(row0 + i * CH, CH), :],
                             sem_out[k])

            pltpu.make_async_copy(
                pe_b[k], pos_hbm.at[pl.ds(row0 + i * CH, CH), :],
                sem_out[k]).wait()

            @pl.when(g < G - 1)
            def _prefetch():
                start_in(i + NSLOT, k)
        return carry

    lax.fori_loop(0, G, outer, 0)

    for k in range(NSLOT):
        i = (G - 1) * NSLOT + k
        pltpu.make_async_copy(o_b[k], out_hbm.at[pl.ds(row0 + i * CH, CH), :],
                              sem_out[k]).wait()


def _body_wrap(x_hbm, idx_hbm, pe_hbm, out_hbm, pos_hbm,
               idx_v, pe0, pe1, pe2, pe3, x0, x1, x2, x3,
               o0, o1, o2, o3, si0, si1, si2, si3, so0, so1, so2, so3):
    _body(x_hbm, idx_hbm, pe_hbm, out_hbm, pos_hbm, idx_v,
          (pe0, pe1, pe2, pe3), (x0, x1, x2, x3), (o0, o1, o2, o3),
          (si0, si1, si2, si3), (so0, so1, so2, so3))


@jax.jit
def _sc_call(x2d, idx, pe):
    mesh = plsc.VectorSubcoreMesh(core_axis_name="c", subcore_axis_name="s")
    buf = pltpu.VMEM((CH, D_MODEL), jnp.float32)
    return pl.kernel(
        _body_wrap,
        out_type=(
            jax.ShapeDtypeStruct((ROWS, D_MODEL), jnp.float32),
            jax.ShapeDtypeStruct((ROWS, D_MODEL), jnp.float32),
        ),
        mesh=mesh,
        scratch_types=(
            [pltpu.VMEM((ROWS_PER_W,), jnp.int32)]
            + [buf] * (3 * NSLOT)
            + [pltpu.SemaphoreType.DMA] * (2 * NSLOT)
        ),
    )(x2d, idx, pe)


def kernel(x, offset, pe):
    assert x.shape == (BATCH, SEQ, D_MODEL)
    x2d = x.reshape(ROWS, D_MODEL)
    index = offset[:, None].astype(jnp.int32) + jnp.arange(SEQ, dtype=jnp.int32)
    index = jnp.maximum(index, 0).reshape(ROWS)
    out_f, pos_f = _sc_call(x2d, index, pe)
    return out_f.reshape(x.shape), pos_f.reshape(x.shape)
